# Initial kernel scaffold; baseline (speedup 1.0000x reference)
import jax
import jax.numpy as jnp
from jax.experimental import pallas as pl


def kernel(entity_indices, entity_embeddings):
    raise NotImplementedError("write your pallas kernel here")


# speedup vs baseline: 1.5729x; 1.5729x over previous
import functools

import jax
import jax.numpy as jnp
from jax import lax
from jax.experimental import pallas as pl
from jax.experimental.pallas import tpu as pltpu
from jax.experimental.pallas import tpu_sc as plsc

B = 16384 * 26
D = 32
NC = 2
NS = 16
NW = NC * NS
B_PER_W = B // NW
G = 128
SPC = 13
CHUNK = G * SPC
NCHUNK = B_PER_W // CHUNK


def _sc_gather(table, idx):
    mesh = plsc.VectorSubcoreMesh(core_axis_name="c", subcore_axis_name="s")

    @functools.partial(
        pl.kernel,
        mesh=mesh,
        out_type=jax.ShapeDtypeStruct((B, D), jnp.float32),
        compiler_params=pltpu.CompilerParams(use_tc_tiling_on_sc=False),
        scratch_types=[
            pltpu.VMEM((NCHUNK * SPC, G), jnp.int32),
            pltpu.VMEM((CHUNK, D), jnp.float32),
            pltpu.VMEM((CHUNK, D), jnp.float32),
            pltpu.SemaphoreType.DMA,
            pltpu.SemaphoreType.DMA,
            pltpu.SemaphoreType.DMA,
            pltpu.SemaphoreType.DMA,
        ],
    )
    def k(table_hbm, idx_hbm, out_hbm, idx_v, buf0, buf1, g0, g1, s0, s1):
        wid = lax.axis_index("s") * NC + lax.axis_index("c")
        base = wid * B_PER_W
        pltpu.sync_copy(idx_hbm.at[wid], idx_v)

        bufs = (buf0, buf1)
        gsems = (g0, g1)
        ssems = (s0, s1)

        def start_gather(c, buf, sem):
            return [
                pltpu.async_copy(
                    table_hbm.at[idx_v.at[c * SPC + t]],
                    buf.at[pl.ds(t * G, G)],
                    sem,
                )
                for t in range(SPC)
            ]

        gather_h = [None] * NCHUNK
        store_h = [None] * NCHUNK

        gather_h[0] = start_gather(0, bufs[0], gsems[0])
        for c in range(NCHUNK):
            b = c & 1
            nb = (c + 1) & 1
            if c + 1 < NCHUNK:
                if c >= 1:
                    store_h[c - 1].wait()
                gather_h[c + 1] = start_gather(c + 1, bufs[nb], gsems[nb])
            for h in gather_h[c]:
                h.wait()
            store_h[c] = pltpu.async_copy(
                bufs[b], out_hbm.at[pl.ds(base + c * CHUNK, CHUNK)], ssems[b])
        store_h[NCHUNK - 2].wait()
        store_h[NCHUNK - 1].wait()

    return k(table, idx)


def kernel(entity_indices, entity_embeddings):
    idx = entity_indices.reshape(NW, NCHUNK * SPC, G)
    out = _sc_gather(entity_embeddings, idx)
    return out.reshape(entity_indices.shape[0], entity_indices.shape[1], D)
